# Initial kernel scaffold; baseline (speedup 1.0000x reference)
import jax
import jax.numpy as jnp
from jax.experimental import pallas as pl


def kernel(x, edge_index_0, edge_index_1, W_enc0, b_enc0, W_enc1, b_enc1, a, W_dec, b_dec):
    raise NotImplementedError("write your pallas kernel here")


# speedup vs baseline: 10.5503x; 10.5503x over previous
import functools

import jax
import jax.numpy as jnp
from jax import lax
from jax.experimental import pallas as pl
from jax.experimental.pallas import tpu as pltpu
from jax.experimental.pallas import tpu_sc as plsc

_N = 10000
_D = 128
_E = 320000

_NC = 2
_NS = 16
_HALF = _D // 2

_CHUNK = 1024
_IDX_ROWS = _CHUNK // 128
_EPT = 20480
_E_PAD = _EPT * _NS
_STEPS = _EPT // _CHUNK
_NPT = _N // _NS
_ACC_ROWS = _N + 16

_HIST_ROWS = _N // 16
_HIST_PAD = _HIST_ROWS + 8

_BN = 2000


def _deg_body(c0_2d, c1_2d, h0_out, h1_out, colbuf, hist):
    c = lax.axis_index("c")
    s = lax.axis_index("s")
    ones = jnp.full((16,), 1.0, dtype=jnp.float32)

    def run(col2d, hout):
        def zero_row(i, carry):
            hist[i] = jnp.zeros((16,), dtype=jnp.float32)
            return carry
        lax.fori_loop(0, _HIST_PAD, zero_row, 0)

        def step(t, carry):
            base = s * (_EPT // 128) + t * 16
            pltpu.sync_copy(col2d.at[pl.ds(base, 16)], colbuf)
            for j in range(16):
                for k in range(8):
                    v = colbuf[j, pl.ds(k * 16, 16)]
                    r = lax.shift_right_logical(v, 4)
                    cc = lax.bitwise_and(v, 15)
                    plsc.addupdate_scatter(hist, [r, cc], ones)
            return carry
        lax.fori_loop(0, _EPT // (16 * 128), step, 0)
        pltpu.sync_copy(hist.at[pl.ds(0, _HIST_ROWS)], hout.at[s])

    @pl.when(c == 0)
    def _():
        run(c0_2d, h0_out)

    @pl.when(c == 1)
    def _():
        run(c1_2d, h1_out)


def _deg_call(c0_2d, c1_2d):
    mesh = plsc.VectorSubcoreMesh(core_axis_name="c", subcore_axis_name="s", num_cores=_NC, num_subcores=_NS)
    f = pl.kernel(
        _deg_body,
        out_type=[
            jax.ShapeDtypeStruct((_NS, _HIST_ROWS, 16), jnp.float32),
            jax.ShapeDtypeStruct((_NS, _HIST_ROWS, 16), jnp.float32),
        ],
        mesh=mesh,
        scratch_types=[
            pltpu.VMEM((16, 128), jnp.int32),
            pltpu.VMEM((_HIST_PAD, 16), jnp.float32),
        ],
        compiler_params=pltpu.CompilerParams(needs_layout_passes=False),
    )
    return f(c0_2d, c1_2d)


def _prop_body(y0lo, y0hi, y1lo, y1hi, r0_2d, c0_2d, r1_2d, c1_2d,
               o0lo, o0hi, o1lo, o1hi,
               colbuf, rowbuf, rowsbuf, acc, gsem, ssem):
    c = lax.axis_index("c")
    s = lax.axis_index("s")

    def run_rel(y_hbm, out_hbm, acc, r2d, c2d):
        pltpu.sync_copy(y_hbm.at[pl.ds(s * _NPT, _NPT)],
                        acc.at[pl.ds(s * _NPT, _NPT)])
        plsc.subcore_barrier()

        def step(t, carry):
            base = s * (_EPT // 128) + t * _IDX_ROWS
            pltpu.sync_copy(c2d.at[pl.ds(base, _IDX_ROWS)], colbuf)
            pltpu.sync_copy(r2d.at[pl.ds(base, _IDX_ROWS)], rowbuf)
            gh = [
                pltpu.async_copy(y_hbm.at[colbuf.at[j]],
                                 rowsbuf.at[pl.ds(j * 128, 128)], gsem)
                for j in range(_IDX_ROWS)
            ]
            for h in gh:
                h.wait()
            sh = [
                pltpu.async_copy(rowsbuf.at[pl.ds(j * 128, 128)],
                                 acc.at[rowbuf.at[j]], ssem, add=True)
                for j in range(_IDX_ROWS)
            ]
            for h in sh:
                h.wait()
            return carry

        lax.fori_loop(0, _STEPS, step, 0)
        plsc.subcore_barrier()
        pltpu.sync_copy(acc.at[pl.ds(s * _NPT, _NPT)],
                        out_hbm.at[pl.ds(s * _NPT, _NPT)])

    @pl.when(c == 0)
    def _():
        run_rel(y0lo, o0lo, acc, r0_2d, c0_2d)
        run_rel(y1lo, o1lo, acc, r1_2d, c1_2d)

    @pl.when(c == 1)
    def _():
        run_rel(y0hi, o0hi, acc, r0_2d, c0_2d)
        run_rel(y1hi, o1hi, acc, r1_2d, c1_2d)


def _prop_call(y0lo, y0hi, y1lo, y1hi, r0_2d, c0_2d, r1_2d, c1_2d):
    mesh = plsc.VectorSubcoreMesh(core_axis_name="c", subcore_axis_name="s", num_cores=_NC, num_subcores=_NS)
    half = jax.ShapeDtypeStruct((_N, _HALF), jnp.float32)
    f = pl.kernel(
        _prop_body,
        out_type=[half, half, half, half],
        mesh=mesh,
        scratch_types=[
            pltpu.VMEM((_IDX_ROWS, 128), jnp.int32),
            pltpu.VMEM((_IDX_ROWS, 128), jnp.int32),
            pltpu.VMEM((_CHUNK, _HALF), jnp.float32),
            pltpu.VMEM_SHARED((_ACC_ROWS, _HALF), jnp.float32),
            pltpu.SemaphoreType.DMA,
            pltpu.SemaphoreType.DMA,
        ],
        compiler_params=pltpu.CompilerParams(use_tc_tiling_on_sc=False),
    )
    return f(y0lo, y0hi, y1lo, y1hi, r0_2d, c0_2d, r1_2d, c1_2d)


def _prep_body(h0_ref, h1_ref, x_ref,
               y0lo, y0hi, y1lo, y1hi, d0_ref, d1_ref):
    deg0 = 1.0 + jnp.sum(h0_ref[...], axis=1, keepdims=True)
    deg1 = 1.0 + jnp.sum(h1_ref[...], axis=1, keepdims=True)
    dinv0 = jnp.where(deg0 > 0, lax.rsqrt(deg0), 0.0)
    dinv1 = jnp.where(deg1 > 0, lax.rsqrt(deg1), 0.0)
    x = x_ref[...]
    y0 = x * dinv0
    y1 = x * dinv1
    y0lo[...] = y0[:, :_HALF]
    y0hi[...] = y0[:, _HALF:]
    y1lo[...] = y1[:, :_HALF]
    y1hi[...] = y1[:, _HALF:]
    d0_ref[...] = dinv0
    d1_ref[...] = dinv1


def _prep_call(hp0, hp1, x):
    grid = (_N // _BN,)
    half_spec = pl.BlockSpec((_BN, _HALF), lambda i: (i, 0))
    f = pl.pallas_call(
        _prep_body,
        grid=grid,
        in_specs=[
            pl.BlockSpec((_BN, 16), lambda i: (i, 0)),
            pl.BlockSpec((_BN, 16), lambda i: (i, 0)),
            pl.BlockSpec((_BN, _D), lambda i: (i, 0)),
        ],
        out_specs=[
            half_spec, half_spec, half_spec, half_spec,
            pl.BlockSpec((_BN, 1), lambda i: (i, 0)),
            pl.BlockSpec((_BN, 1), lambda i: (i, 0)),
        ],
        out_shape=[
            jax.ShapeDtypeStruct((_N, _HALF), jnp.float32),
            jax.ShapeDtypeStruct((_N, _HALF), jnp.float32),
            jax.ShapeDtypeStruct((_N, _HALF), jnp.float32),
            jax.ShapeDtypeStruct((_N, _HALF), jnp.float32),
            jax.ShapeDtypeStruct((_N, 1), jnp.float32),
            jax.ShapeDtypeStruct((_N, 1), jnp.float32),
        ],
    )
    return f(hp0, hp1, x)


def _mid_body(o0lo, o0hi, o1lo, o1hi, d0_ref, d1_ref,
              z0lo, z0hi, z1lo, z1hi):
    s0 = d0_ref[...] * d0_ref[...]
    s1 = d1_ref[...] * d1_ref[...]
    z0lo[...] = o0lo[...] * s0
    z0hi[...] = o0hi[...] * s0
    z1lo[...] = o1lo[...] * s1
    z1hi[...] = o1hi[...] * s1


def _mid_call(o0lo, o0hi, o1lo, o1hi, d0, d1):
    grid = (_N // _BN,)
    half_spec = pl.BlockSpec((_BN, _HALF), lambda i: (i, 0))
    dspec = pl.BlockSpec((_BN, 1), lambda i: (i, 0))
    half = jax.ShapeDtypeStruct((_N, _HALF), jnp.float32)
    f = pl.pallas_call(
        _mid_body,
        grid=grid,
        in_specs=[half_spec, half_spec, half_spec, half_spec, dspec, dspec],
        out_specs=[half_spec, half_spec, half_spec, half_spec],
        out_shape=[half, half, half, half],
    )
    return f(o0lo, o0hi, o1lo, o1hi, d0, d1)


def _final_body(w0lo, w0hi, w1lo, w1hi, d0_ref, d1_ref,
                W0_ref, b0_ref, W1_ref, b1_ref, a_ref, Wd_ref, bd_ref,
                out_ref):
    h0 = jnp.concatenate([w0lo[...], w0hi[...]], axis=1) * d0_ref[...]
    h1 = jnp.concatenate([w1lo[...], w1hi[...]], axis=1) * d1_ref[...]
    e0 = jnp.dot(h0, W0_ref[...], preferred_element_type=jnp.float32,
                 precision=lax.Precision.HIGHEST) + b0_ref[...]
    e1 = jnp.dot(h1, W1_ref[...], preferred_element_type=jnp.float32,
                 precision=lax.Precision.HIGHEST) + b1_ref[...]
    a0 = a_ref[0, 0]
    a1 = a_ref[0, 1]
    m = jnp.maximum(a0, a1)
    x0 = jnp.exp(a0 - m)
    x1 = jnp.exp(a1 - m)
    ws0 = x0 / (x0 + x1)
    ws1 = x1 / (x0 + x1)
    fused = ws0 * e0 + ws1 * e1
    out_ref[...] = jnp.dot(fused, Wd_ref[...], preferred_element_type=jnp.float32,
                           precision=lax.Precision.HIGHEST) + bd_ref[...]


def _final_call(w0lo, w0hi, w1lo, w1hi, d0, d1, W0, b0, W1, b1, a2d, Wd, bd):
    grid = (_N // _BN,)
    half_spec = pl.BlockSpec((_BN, _HALF), lambda i: (i, 0))
    dspec = pl.BlockSpec((_BN, 1), lambda i: (i, 0))
    wspec = pl.BlockSpec((_D, _D), lambda i: (0, 0))
    bspec = pl.BlockSpec((1, _D), lambda i: (0, 0))
    f = pl.pallas_call(
        _final_body,
        grid=grid,
        in_specs=[
            half_spec, half_spec, half_spec, half_spec, dspec, dspec,
            wspec, bspec, wspec, bspec,
            pl.BlockSpec((1, 2), lambda i: (0, 0)),
            wspec, bspec,
        ],
        out_specs=pl.BlockSpec((_BN, _D), lambda i: (i, 0)),
        out_shape=jax.ShapeDtypeStruct((_N, _D), jnp.float32),
    )
    return f(w0lo, w0hi, w1lo, w1hi, d0, d1, W0, b0, W1, b1, a2d, Wd, bd)


def _pad_idx(idx, fill):
    pad = jnp.full((_E_PAD - _E,), fill, dtype=jnp.int32)
    return jnp.concatenate([idx.astype(jnp.int32), pad]).reshape(_E_PAD // 128, 128)


@jax.jit
def kernel(x, edge_index_0, edge_index_1, W_enc0, b_enc0, W_enc1, b_enc1,
           a, W_dec, b_dec):
    r0 = _pad_idx(edge_index_0[0], _N)
    c0 = _pad_idx(edge_index_0[1], 0)
    r1 = _pad_idx(edge_index_1[0], _N)
    c1 = _pad_idx(edge_index_1[1], 0)
    c0d = _pad_idx(edge_index_0[1], _N)
    c1d = _pad_idx(edge_index_1[1], _N)

    hp0, hp1 = _deg_call(c0d, c1d)
    hp0 = jnp.transpose(hp0, (1, 2, 0)).reshape(_N, _NS)
    hp1 = jnp.transpose(hp1, (1, 2, 0)).reshape(_N, _NS)

    y0lo, y0hi, y1lo, y1hi, d0, d1 = _prep_call(hp0, hp1, x)

    g0lo, g0hi, g1lo, g1hi = _prop_call(y0lo, y0hi, y1lo, y1hi, r0, c0, r1, c1)

    z0lo, z0hi, z1lo, z1hi = _mid_call(g0lo, g0hi, g1lo, g1hi, d0, d1)

    w0lo, w0hi, w1lo, w1hi = _prop_call(z0lo, z0hi, z1lo, z1hi, r0, c0, r1, c1)

    a2d = a.reshape(1, 2).astype(jnp.float32)
    return _final_call(w0lo, w0hi, w1lo, w1hi, d0, d1,
                       W_enc0, b_enc0.reshape(1, _D),
                       W_enc1, b_enc1.reshape(1, _D),
                       a2d, W_dec, b_dec.reshape(1, _D))


# speedup vs baseline: 18.8440x; 1.7861x over previous
import functools

import jax
import jax.numpy as jnp
from jax import lax
from jax.experimental import pallas as pl
from jax.experimental.pallas import tpu as pltpu
from jax.experimental.pallas import tpu_sc as plsc

_N = 10000
_D = 128
_E = 320000

_NC = 2
_NS = 16
_HALF = _D // 2

_CHUNK = 512
_IDX_ROWS = _CHUNK // 128
_EPT = 20480
_E_PAD = _EPT * _NS
_STEPS = _EPT // _CHUNK
_NPT = _N // _NS
_ACC_ROWS = _N + 16

_HIST_ROWS = _N // 16
_HIST_PAD = _HIST_ROWS + 8

_BN = 2000


def _deg_body(c0_2d, c1_2d, h0_out, h1_out, colbuf, hist):
    c = lax.axis_index("c")
    s = lax.axis_index("s")
    ones = jnp.full((16,), 1.0, dtype=jnp.float32)

    def run(col2d, hout):
        def zero_row(i, carry):
            hist[i] = jnp.zeros((16,), dtype=jnp.float32)
            return carry
        lax.fori_loop(0, _HIST_PAD, zero_row, 0)

        def step(t, carry):
            base = s * (_EPT // 128) + t * 16
            pltpu.sync_copy(col2d.at[pl.ds(base, 16)], colbuf)
            for j in range(16):
                for k in range(8):
                    v = colbuf[j, pl.ds(k * 16, 16)]
                    r = lax.shift_right_logical(v, 4)
                    cc = lax.bitwise_and(v, 15)
                    plsc.addupdate_scatter(hist, [r, cc], ones)
            return carry
        lax.fori_loop(0, _EPT // (16 * 128), step, 0)
        pltpu.sync_copy(hist.at[pl.ds(0, _HIST_ROWS)], hout.at[s])

    @pl.when(c == 0)
    def _():
        run(c0_2d, h0_out)

    @pl.when(c == 1)
    def _():
        run(c1_2d, h1_out)


def _deg_call(c0_2d, c1_2d):
    mesh = plsc.VectorSubcoreMesh(core_axis_name="c", subcore_axis_name="s", num_cores=_NC, num_subcores=_NS)
    f = pl.kernel(
        _deg_body,
        out_type=[
            jax.ShapeDtypeStruct((_NS, _HIST_ROWS, 16), jnp.float32),
            jax.ShapeDtypeStruct((_NS, _HIST_ROWS, 16), jnp.float32),
        ],
        mesh=mesh,
        scratch_types=[
            pltpu.VMEM((16, 128), jnp.int32),
            pltpu.VMEM((_HIST_PAD, 16), jnp.float32),
        ],
        compiler_params=pltpu.CompilerParams(needs_layout_passes=False),
    )
    return f(c0_2d, c1_2d)


def _prop_body(y0lo, y0hi, y1lo, y1hi, r0_2d, c0_2d, r1_2d, c1_2d,
               o0lo, o0hi, o1lo, o1hi,
               colbuf, rowbuf, rowsbuf, ysp, acc, gsem, ssem):
    c = lax.axis_index("c")
    s = lax.axis_index("s")

    def run_rel(y_hbm, out_hbm, acc, r2d, c2d):
        pltpu.sync_copy(y_hbm.at[pl.ds(s * _NPT, _NPT)],
                        ysp.at[pl.ds(s * _NPT, _NPT)])
        pltpu.sync_copy(y_hbm.at[pl.ds(s * _NPT, _NPT)],
                        acc.at[pl.ds(s * _NPT, _NPT)])
        plsc.subcore_barrier()

        def step(t, carry):
            base = s * (_EPT // 128) + t * _IDX_ROWS
            pltpu.sync_copy(c2d.at[pl.ds(base, _IDX_ROWS)], colbuf)
            pltpu.sync_copy(r2d.at[pl.ds(base, _IDX_ROWS)], rowbuf)
            gh = [
                pltpu.async_copy(ysp.at[colbuf.at[j]],
                                 rowsbuf.at[pl.ds(j * 128, 128)], gsem)
                for j in range(_IDX_ROWS)
            ]
            for h in gh:
                h.wait()
            sh = [
                pltpu.async_copy(rowsbuf.at[pl.ds(j * 128, 128)],
                                 acc.at[rowbuf.at[j]], ssem, add=True)
                for j in range(_IDX_ROWS)
            ]
            for h in sh:
                h.wait()
            return carry

        lax.fori_loop(0, _STEPS, step, 0)
        plsc.subcore_barrier()
        pltpu.sync_copy(acc.at[pl.ds(s * _NPT, _NPT)],
                        out_hbm.at[pl.ds(s * _NPT, _NPT)])

    @pl.when(c == 0)
    def _():
        run_rel(y0lo, o0lo, acc, r0_2d, c0_2d)
        run_rel(y1lo, o1lo, acc, r1_2d, c1_2d)

    @pl.when(c == 1)
    def _():
        run_rel(y0hi, o0hi, acc, r0_2d, c0_2d)
        run_rel(y1hi, o1hi, acc, r1_2d, c1_2d)


def _prop_call(y0lo, y0hi, y1lo, y1hi, r0_2d, c0_2d, r1_2d, c1_2d):
    mesh = plsc.VectorSubcoreMesh(core_axis_name="c", subcore_axis_name="s", num_cores=_NC, num_subcores=_NS)
    half = jax.ShapeDtypeStruct((_N, _HALF), jnp.float32)
    f = pl.kernel(
        _prop_body,
        out_type=[half, half, half, half],
        mesh=mesh,
        scratch_types=[
            pltpu.VMEM((_IDX_ROWS, 128), jnp.int32),
            pltpu.VMEM((_IDX_ROWS, 128), jnp.int32),
            pltpu.VMEM((_CHUNK, _HALF), jnp.float32),
            pltpu.VMEM_SHARED((_N, _HALF), jnp.float32),
            pltpu.VMEM_SHARED((_ACC_ROWS, _HALF), jnp.float32),
            pltpu.SemaphoreType.DMA,
            pltpu.SemaphoreType.DMA,
        ],
        compiler_params=pltpu.CompilerParams(use_tc_tiling_on_sc=False),
    )
    return f(y0lo, y0hi, y1lo, y1hi, r0_2d, c0_2d, r1_2d, c1_2d)


def _prep_body(h0_ref, h1_ref, x_ref,
               y0lo, y0hi, y1lo, y1hi, d0_ref, d1_ref):
    deg0 = 1.0 + jnp.sum(h0_ref[...], axis=1, keepdims=True)
    deg1 = 1.0 + jnp.sum(h1_ref[...], axis=1, keepdims=True)
    dinv0 = jnp.where(deg0 > 0, lax.rsqrt(deg0), 0.0)
    dinv1 = jnp.where(deg1 > 0, lax.rsqrt(deg1), 0.0)
    x = x_ref[...]
    y0 = x * dinv0
    y1 = x * dinv1
    y0lo[...] = y0[:, :_HALF]
    y0hi[...] = y0[:, _HALF:]
    y1lo[...] = y1[:, :_HALF]
    y1hi[...] = y1[:, _HALF:]
    d0_ref[...] = dinv0
    d1_ref[...] = dinv1


def _prep_call(hp0, hp1, x):
    grid = (_N // _BN,)
    half_spec = pl.BlockSpec((_BN, _HALF), lambda i: (i, 0))
    f = pl.pallas_call(
        _prep_body,
        grid=grid,
        in_specs=[
            pl.BlockSpec((_BN, 16), lambda i: (i, 0)),
            pl.BlockSpec((_BN, 16), lambda i: (i, 0)),
            pl.BlockSpec((_BN, _D), lambda i: (i, 0)),
        ],
        out_specs=[
            half_spec, half_spec, half_spec, half_spec,
            pl.BlockSpec((_BN, 1), lambda i: (i, 0)),
            pl.BlockSpec((_BN, 1), lambda i: (i, 0)),
        ],
        out_shape=[
            jax.ShapeDtypeStruct((_N, _HALF), jnp.float32),
            jax.ShapeDtypeStruct((_N, _HALF), jnp.float32),
            jax.ShapeDtypeStruct((_N, _HALF), jnp.float32),
            jax.ShapeDtypeStruct((_N, _HALF), jnp.float32),
            jax.ShapeDtypeStruct((_N, 1), jnp.float32),
            jax.ShapeDtypeStruct((_N, 1), jnp.float32),
        ],
    )
    return f(hp0, hp1, x)


def _mid_body(o0lo, o0hi, o1lo, o1hi, d0_ref, d1_ref,
              z0lo, z0hi, z1lo, z1hi):
    s0 = d0_ref[...] * d0_ref[...]
    s1 = d1_ref[...] * d1_ref[...]
    z0lo[...] = o0lo[...] * s0
    z0hi[...] = o0hi[...] * s0
    z1lo[...] = o1lo[...] * s1
    z1hi[...] = o1hi[...] * s1


def _mid_call(o0lo, o0hi, o1lo, o1hi, d0, d1):
    grid = (_N // _BN,)
    half_spec = pl.BlockSpec((_BN, _HALF), lambda i: (i, 0))
    dspec = pl.BlockSpec((_BN, 1), lambda i: (i, 0))
    half = jax.ShapeDtypeStruct((_N, _HALF), jnp.float32)
    f = pl.pallas_call(
        _mid_body,
        grid=grid,
        in_specs=[half_spec, half_spec, half_spec, half_spec, dspec, dspec],
        out_specs=[half_spec, half_spec, half_spec, half_spec],
        out_shape=[half, half, half, half],
    )
    return f(o0lo, o0hi, o1lo, o1hi, d0, d1)


def _final_body(w0lo, w0hi, w1lo, w1hi, d0_ref, d1_ref,
                W0_ref, b0_ref, W1_ref, b1_ref, a_ref, Wd_ref, bd_ref,
                out_ref):
    h0 = jnp.concatenate([w0lo[...], w0hi[...]], axis=1) * d0_ref[...]
    h1 = jnp.concatenate([w1lo[...], w1hi[...]], axis=1) * d1_ref[...]
    e0 = jnp.dot(h0, W0_ref[...], preferred_element_type=jnp.float32,
                 precision=lax.Precision.HIGHEST) + b0_ref[...]
    e1 = jnp.dot(h1, W1_ref[...], preferred_element_type=jnp.float32,
                 precision=lax.Precision.HIGHEST) + b1_ref[...]
    a0 = a_ref[0, 0]
    a1 = a_ref[0, 1]
    m = jnp.maximum(a0, a1)
    x0 = jnp.exp(a0 - m)
    x1 = jnp.exp(a1 - m)
    ws0 = x0 / (x0 + x1)
    ws1 = x1 / (x0 + x1)
    fused = ws0 * e0 + ws1 * e1
    out_ref[...] = jnp.dot(fused, Wd_ref[...], preferred_element_type=jnp.float32,
                           precision=lax.Precision.HIGHEST) + bd_ref[...]


def _final_call(w0lo, w0hi, w1lo, w1hi, d0, d1, W0, b0, W1, b1, a2d, Wd, bd):
    grid = (_N // _BN,)
    half_spec = pl.BlockSpec((_BN, _HALF), lambda i: (i, 0))
    dspec = pl.BlockSpec((_BN, 1), lambda i: (i, 0))
    wspec = pl.BlockSpec((_D, _D), lambda i: (0, 0))
    bspec = pl.BlockSpec((1, _D), lambda i: (0, 0))
    f = pl.pallas_call(
        _final_body,
        grid=grid,
        in_specs=[
            half_spec, half_spec, half_spec, half_spec, dspec, dspec,
            wspec, bspec, wspec, bspec,
            pl.BlockSpec((1, 2), lambda i: (0, 0)),
            wspec, bspec,
        ],
        out_specs=pl.BlockSpec((_BN, _D), lambda i: (i, 0)),
        out_shape=jax.ShapeDtypeStruct((_N, _D), jnp.float32),
    )
    return f(w0lo, w0hi, w1lo, w1hi, d0, d1, W0, b0, W1, b1, a2d, Wd, bd)


def _pad_idx(idx, fill):
    pad = jnp.full((_E_PAD - _E,), fill, dtype=jnp.int32)
    return jnp.concatenate([idx.astype(jnp.int32), pad]).reshape(_E_PAD // 128, 128)


@jax.jit
def kernel(x, edge_index_0, edge_index_1, W_enc0, b_enc0, W_enc1, b_enc1,
           a, W_dec, b_dec):
    r0 = _pad_idx(edge_index_0[0], _N)
    c0 = _pad_idx(edge_index_0[1], 0)
    r1 = _pad_idx(edge_index_1[0], _N)
    c1 = _pad_idx(edge_index_1[1], 0)
    c0d = _pad_idx(edge_index_0[1], _N)
    c1d = _pad_idx(edge_index_1[1], _N)

    hp0, hp1 = _deg_call(c0d, c1d)
    hp0 = jnp.transpose(hp0, (1, 2, 0)).reshape(_N, _NS)
    hp1 = jnp.transpose(hp1, (1, 2, 0)).reshape(_N, _NS)

    y0lo, y0hi, y1lo, y1hi, d0, d1 = _prep_call(hp0, hp1, x)

    g0lo, g0hi, g1lo, g1hi = _prop_call(y0lo, y0hi, y1lo, y1hi, r0, c0, r1, c1)

    z0lo, z0hi, z1lo, z1hi = _mid_call(g0lo, g0hi, g1lo, g1hi, d0, d1)

    w0lo, w0hi, w1lo, w1hi = _prop_call(z0lo, z0hi, z1lo, z1hi, r0, c0, r1, c1)

    a2d = a.reshape(1, 2).astype(jnp.float32)
    return _final_call(w0lo, w0hi, w1lo, w1hi, d0, d1,
                       W_enc0, b_enc0.reshape(1, _D),
                       W_enc1, b_enc1.reshape(1, _D),
                       a2d, W_dec, b_dec.reshape(1, _D))
